# baseline (speedup 1.0000x reference)
import functools

import jax
import jax.numpy as jnp
from jax import lax
from jax.experimental import pallas as pl
from jax.experimental.pallas import tpu as pltpu
from jax.experimental.pallas import tpu_sc as plsc

S = 4096
M = 1024
E = 16
C = 512
F = 4096
NSLOT = (E + 1) * C
TRASH = E * C
ZROW = 0
NW = 32
FBLK = 2048
FSTEPS = F // FBLK


def _router_body(x_ref, wg_ref, s0_ref, s1_ref, c0_ref, c1_ref,
                 w0_ref, w1_ref, va_ref, vb_ref, ll_ref):
    x = x_ref[...]
    logits = lax.dot_general(x, wg_ref[...], (((1,), (1,)), ((), ())),
                             preferred_element_type=jnp.float32)
    iota = lax.broadcasted_iota(jnp.int32, (S, E), 1)
    riota = lax.broadcasted_iota(jnp.int32, (S, E), 0)

    m0 = jnp.max(logits, axis=1, keepdims=True)
    i0 = jnp.min(jnp.where(logits == m0, iota, E), axis=1, keepdims=True)
    mask0b = iota == i0
    l2 = jnp.where(mask0b, -jnp.inf, logits)
    m1 = jnp.max(l2, axis=1, keepdims=True)
    i1 = jnp.min(jnp.where(l2 == m1, iota, E), axis=1, keepdims=True)
    mask1b = iota == i1

    eg = jnp.exp(logits - m0)
    gates = eg / jnp.sum(eg, axis=1, keepdims=True)
    g0 = jnp.sum(jnp.where(mask0b, gates, 0.0), axis=1, keepdims=True)
    g1 = jnp.sum(jnp.where(mask1b, gates, 0.0), axis=1, keepdims=True)
    denom = jnp.maximum(g0 + g1, jnp.finfo(jnp.float32).eps)

    mask0 = mask0b.astype(jnp.float32)
    mask1 = mask1b.astype(jnp.float32)

    def cum(m):
        c = m
        sh = 1
        while sh < S:
            r = pltpu.roll(c, sh, 0)
            c = c + jnp.where(riota >= sh, r, 0.0)
            sh *= 2
        return c

    c0 = cum(mask0)
    count0 = c0[S - 1:S, :]
    c1 = cum(mask1)
    loc0 = jnp.sum((c0 - 1.0) * mask0, axis=1, keepdims=True)
    loc1 = jnp.sum((c1 - 1.0 + count0) * mask1, axis=1, keepdims=True)
    loc0i = loc0.astype(jnp.int32)
    loc1i = loc1.astype(jnp.int32)
    v0 = loc0i < C
    v1 = loc1i < C

    s0_ref[...] = jnp.where(v0, i0 * C + loc0i, TRASH)
    s1_ref[...] = jnp.where(v1, i1 * C + loc1i, TRASH)
    c0_ref[...] = jnp.where(v0, i0 * C + loc0i, 0)
    c1_ref[...] = jnp.where(v1, i1 * C + loc1i, 0)
    w0_ref[...] = jnp.where(v0, g0 / denom, 0.0)
    w1_ref[...] = jnp.where(v1, g1 / denom, 0.0)
    va_ref[...] = v0.astype(jnp.float32)
    vb_ref[...] = v1.astype(jnp.float32)

    me = jnp.sum(gates, axis=0, keepdims=True)
    ll_ref[...] = jnp.sum(me * count0, axis=1, keepdims=True) * (E / (S * S))


_router = pl.pallas_call(
    _router_body,
    out_shape=(
        jax.ShapeDtypeStruct((S, 1), jnp.int32),
        jax.ShapeDtypeStruct((S, 1), jnp.int32),
        jax.ShapeDtypeStruct((S, 1), jnp.int32),
        jax.ShapeDtypeStruct((S, 1), jnp.int32),
        jax.ShapeDtypeStruct((S, 1), jnp.float32),
        jax.ShapeDtypeStruct((S, 1), jnp.float32),
        jax.ShapeDtypeStruct((S, 1), jnp.float32),
        jax.ShapeDtypeStruct((S, 1), jnp.float32),
        jax.ShapeDtypeStruct((1, 1), jnp.float32),
    ),
)


def _scatter_call(slots, wts):
    mesh = plsc.VectorSubcoreMesh(core_axis_name="c", subcore_axis_name="s")

    @functools.partial(
        pl.kernel,
        mesh=mesh,
        out_type=(
            jax.ShapeDtypeStruct((NSLOT,), jnp.int32),
            jax.ShapeDtypeStruct((NSLOT,), jnp.float32),
        ),
        scratch_types=[
            pltpu.VMEM((NSLOT,), jnp.int32),
            pltpu.VMEM((NSLOT,), jnp.float32),
            pltpu.VMEM((2 * S,), jnp.int32),
            pltpu.VMEM((2 * S,), jnp.float32),
        ],
        compiler_params=pltpu.CompilerParams(needs_layout_passes=False),
    )
    def k(slots_hbm, wts_hbm, ss_hbm, ws_hbm, sv, wv, slv, wtv):
        wid = lax.axis_index("s") * 2 + lax.axis_index("c")

        @pl.when(wid == 0)
        def _():
            pltpu.sync_copy(slots_hbm, slv)

            def init(i, carry):
                sv[pl.ds(i * 16, 16)] = jnp.full((16,), ZROW, jnp.int32)
                return carry

            lax.fori_loop(0, NSLOT // 16, init, 0)

            def scat(i, carry):
                idx = slv[pl.ds(i * 16, 16)]
                tok = (lax.iota(jnp.int32, 16) + i * 16) & (S - 1)
                plsc.store_scatter(sv, [idx], tok)
                return carry

            lax.fori_loop(0, 2 * S // 16, scat, 0)
            pltpu.sync_copy(sv, ss_hbm)

        @pl.when(wid == 1)
        def _():
            pltpu.sync_copy(slots_hbm, slv)
            pltpu.sync_copy(wts_hbm, wtv)

            def initw(i, carry):
                wv[pl.ds(i * 16, 16)] = jnp.zeros((16,), jnp.float32)
                return carry

            lax.fori_loop(0, NSLOT // 16, initw, 0)

            def scatw(i, carry):
                idx = slv[pl.ds(i * 16, 16)]
                w = wtv[pl.ds(i * 16, 16)]
                plsc.store_scatter(wv, [idx], w)
                return carry

            lax.fori_loop(0, 2 * S // 16, scatw, 0)
            pltpu.sync_copy(wv, ws_hbm)

    return k(slots, wts)


def _gather_call(src, idx, chunk=32):
    n = idx.shape[0]
    per_w = n // NW
    nch = per_w // chunk
    mesh = plsc.VectorSubcoreMesh(core_axis_name="c", subcore_axis_name="s")

    @functools.partial(
        pl.kernel,
        mesh=mesh,
        out_type=jax.ShapeDtypeStruct((n, M), jnp.float32),
        scratch_types=[
            pltpu.VMEM((per_w,), jnp.int32),
            pltpu.VMEM((chunk, M), jnp.float32),
            pltpu.VMEM((chunk, M), jnp.float32),
            pltpu.VMEM((chunk, M), jnp.float32),
            pltpu.SemaphoreType.DMA,
            pltpu.SemaphoreType.DMA,
            pltpu.SemaphoreType.DMA,
            pltpu.SemaphoreType.DMA,
            pltpu.SemaphoreType.DMA,
            pltpu.SemaphoreType.DMA,
        ],
    )
    def k(src_hbm, idx_hbm, out_hbm, idxv, b0, b1, b2, g0, g1, g2, o0, o1, o2):
        wid = lax.axis_index("s") * 2 + lax.axis_index("c")
        base = wid * per_w
        pltpu.sync_copy(idx_hbm.at[pl.ds(base, per_w)], idxv)
        bufs = [(b0, g0, o0), (b1, g1, o1), (b2, g2, o2)]
        nbuf = 3
        gathers = [None] * nch
        writes = [None] * nch
        for g in range(nch):
            b, gsem, _ = bufs[g % nbuf]
            if g >= nbuf:
                writes[g - nbuf].wait()
            gathers[g] = pltpu.async_copy(
                src_hbm.at[idxv.at[pl.ds(g * chunk, chunk)]], b, gsem)
            if g >= 1:
                pb, _, posem = bufs[(g - 1) % nbuf]
                gathers[g - 1].wait()
                writes[g - 1] = pltpu.async_copy(
                    pb, out_hbm.at[pl.ds(base + (g - 1) * chunk, chunk)], posem)
        gathers[nch - 1].wait()
        lb, _, losem = bufs[(nch - 1) % nbuf]
        writes[nch - 1] = pltpu.async_copy(
            lb, out_hbm.at[pl.ds(base + (nch - 1) * chunk, chunk)], losem)
        for t in range(max(0, nch - nbuf), nch):
            writes[t].wait()

    return k(src, idx)


def _ffn_body(d_ref, w1_ref, w2_ref, ws_ref, o_ref):
    f = pl.program_id(1)

    @pl.when(f == 0)
    def _():
        o_ref[...] = jnp.zeros_like(o_ref)

    db = d_ref[...].astype(jnp.bfloat16)
    w1b = w1_ref[0].astype(jnp.bfloat16)
    h = jnp.dot(db, w1b, preferred_element_type=jnp.float32)
    hb = jnp.maximum(h, 0.0).astype(jnp.bfloat16)
    w2b = w2_ref[0].astype(jnp.bfloat16)
    o_ref[...] += jnp.dot(hb, w2b, preferred_element_type=jnp.float32)

    @pl.when(f == FSTEPS - 1)
    def _():
        o_ref[...] *= ws_ref[...]


EH = E // 2


def _ffn_half_body(d_ref, w1_ref, w2_ref, ws_ref, prev_ref, o_ref):
    del prev_ref
    _ffn_body(d_ref, w1_ref, w2_ref, ws_ref, o_ref)


_ffn_lo = pl.pallas_call(
    _ffn_body,
    grid=(EH, FSTEPS),
    in_specs=[
        pl.BlockSpec((C, M), lambda e, f: (e, 0)),
        pl.BlockSpec((1, M, FBLK), lambda e, f: (e, 0, f)),
        pl.BlockSpec((1, FBLK, M), lambda e, f: (e, f, 0)),
        pl.BlockSpec((C, 1), lambda e, f: (e, 0)),
    ],
    out_specs=pl.BlockSpec((C, M), lambda e, f: (e, 0)),
    out_shape=jax.ShapeDtypeStruct((E * C, M), jnp.float32),
    compiler_params=pltpu.CompilerParams(
        dimension_semantics=("parallel", "arbitrary")),
)

_ffn_hi = pl.pallas_call(
    _ffn_half_body,
    grid=(EH, FSTEPS),
    in_specs=[
        pl.BlockSpec((C, M), lambda e, f: (e, 0)),
        pl.BlockSpec((1, M, FBLK), lambda e, f: (e + EH, 0, f)),
        pl.BlockSpec((1, FBLK, M), lambda e, f: (e + EH, f, 0)),
        pl.BlockSpec((C, 1), lambda e, f: (e, 0)),
        pl.BlockSpec((8, 128), lambda e, f: (0, 0)),
    ],
    out_specs=pl.BlockSpec((C, M), lambda e, f: (e + EH, 0)),
    out_shape=jax.ShapeDtypeStruct((E * C, M), jnp.float32),
    input_output_aliases={4: 0},
    compiler_params=pltpu.CompilerParams(
        dimension_semantics=("parallel", "arbitrary")),
)


def _add_body(a_ref, b_ref, va_ref, vb_ref, o_ref):
    o_ref[...] = a_ref[...] * va_ref[...] + b_ref[...] * vb_ref[...]


_combine_add = pl.pallas_call(
    _add_body,
    grid=(S // C,),
    in_specs=[
        pl.BlockSpec((C, M), lambda i: (i, 0)),
        pl.BlockSpec((C, M), lambda i: (i + S // C, 0)),
        pl.BlockSpec((C, 1), lambda i: (i, 0)),
        pl.BlockSpec((C, 1), lambda i: (i, 0)),
    ],
    out_specs=pl.BlockSpec((C, M), lambda i: (i, 0)),
    out_shape=jax.ShapeDtypeStruct((S, M), jnp.float32),
)


def kernel(input, wg, w1, w2):
    x = input
    s0, s1, c0, c1, w0g, w1g, va, vb, ll = _router(x, wg)
    slots = jnp.concatenate([s0.reshape(S), s1.reshape(S)])
    cslots = jnp.concatenate([c0.reshape(S), c1.reshape(S)])
    wts = jnp.concatenate([w0g.reshape(S), w1g.reshape(S)])
    slot_src, wslot = _scatter_call(slots, wts)
    half = EH * C
    disp_lo = _gather_call(x, slot_src[:half])
    disp_hi = _gather_call(x, slot_src[half:2 * half])
    ws2d = wslot[:E * C].reshape(E * C, 1)
    eo_lo = _ffn_lo(disp_lo, w1, w2, ws2d[:half])
    eo = _ffn_hi(disp_hi, w1, w2, ws2d[half:], eo_lo)
    ab = _gather_call(eo, cslots)
    y = _combine_add(ab, ab, va, vb)
    return y, ll.reshape(())


# speedup vs baseline: 1.0158x; 1.0158x over previous
import functools

import jax
import jax.numpy as jnp
from jax import lax
from jax.experimental import pallas as pl
from jax.experimental.pallas import tpu as pltpu
from jax.experimental.pallas import tpu_sc as plsc

S = 4096
M = 1024
E = 16
C = 512
F = 4096
NSLOT = (E + 1) * C
TRASH = E * C
ZROW = 0
NW = 32
FBLK = 2048
FSTEPS = F // FBLK


def _router_body(x_ref, wg_ref, s0_ref, s1_ref, c0_ref, c1_ref,
                 w0_ref, w1_ref, va_ref, vb_ref, ll_ref):
    x = x_ref[...]
    logits = lax.dot_general(x, wg_ref[...], (((1,), (1,)), ((), ())),
                             preferred_element_type=jnp.float32)
    iota = lax.broadcasted_iota(jnp.int32, (S, E), 1)
    riota = lax.broadcasted_iota(jnp.int32, (S, E), 0)

    m0 = jnp.max(logits, axis=1, keepdims=True)
    i0 = jnp.min(jnp.where(logits == m0, iota, E), axis=1, keepdims=True)
    mask0b = iota == i0
    l2 = jnp.where(mask0b, -jnp.inf, logits)
    m1 = jnp.max(l2, axis=1, keepdims=True)
    i1 = jnp.min(jnp.where(l2 == m1, iota, E), axis=1, keepdims=True)
    mask1b = iota == i1

    eg = jnp.exp(logits - m0)
    gates = eg / jnp.sum(eg, axis=1, keepdims=True)
    g0 = jnp.sum(jnp.where(mask0b, gates, 0.0), axis=1, keepdims=True)
    g1 = jnp.sum(jnp.where(mask1b, gates, 0.0), axis=1, keepdims=True)
    denom = jnp.maximum(g0 + g1, jnp.finfo(jnp.float32).eps)

    mask0 = mask0b.astype(jnp.float32)
    mask1 = mask1b.astype(jnp.float32)

    def cum(m):
        c = m
        sh = 1
        while sh < S:
            r = pltpu.roll(c, sh, 0)
            c = c + jnp.where(riota >= sh, r, 0.0)
            sh *= 2
        return c

    c0 = cum(mask0)
    count0 = c0[S - 1:S, :]
    c1 = cum(mask1)
    loc0 = jnp.sum((c0 - 1.0) * mask0, axis=1, keepdims=True)
    loc1 = jnp.sum((c1 - 1.0 + count0) * mask1, axis=1, keepdims=True)
    loc0i = loc0.astype(jnp.int32)
    loc1i = loc1.astype(jnp.int32)
    v0 = loc0i < C
    v1 = loc1i < C

    s0_ref[...] = jnp.where(v0, i0 * C + loc0i, TRASH)
    s1_ref[...] = jnp.where(v1, i1 * C + loc1i, TRASH)
    c0_ref[...] = jnp.where(v0, i0 * C + loc0i, 0)
    c1_ref[...] = jnp.where(v1, i1 * C + loc1i, 0)
    w0_ref[...] = jnp.where(v0, g0 / denom, 0.0)
    w1_ref[...] = jnp.where(v1, g1 / denom, 0.0)
    va_ref[...] = v0.astype(jnp.float32)
    vb_ref[...] = v1.astype(jnp.float32)

    me = jnp.sum(gates, axis=0, keepdims=True)
    ll_ref[...] = jnp.sum(me * count0, axis=1, keepdims=True) * (E / (S * S))


_router = pl.pallas_call(
    _router_body,
    out_shape=(
        jax.ShapeDtypeStruct((S, 1), jnp.int32),
        jax.ShapeDtypeStruct((S, 1), jnp.int32),
        jax.ShapeDtypeStruct((S, 1), jnp.int32),
        jax.ShapeDtypeStruct((S, 1), jnp.int32),
        jax.ShapeDtypeStruct((S, 1), jnp.float32),
        jax.ShapeDtypeStruct((S, 1), jnp.float32),
        jax.ShapeDtypeStruct((S, 1), jnp.float32),
        jax.ShapeDtypeStruct((S, 1), jnp.float32),
        jax.ShapeDtypeStruct((1, 1), jnp.float32),
    ),
)


def _scatter_call(slots, wts):
    mesh = plsc.VectorSubcoreMesh(core_axis_name="c", subcore_axis_name="s")

    @functools.partial(
        pl.kernel,
        mesh=mesh,
        out_type=(
            jax.ShapeDtypeStruct((NSLOT,), jnp.int32),
            jax.ShapeDtypeStruct((NSLOT,), jnp.float32),
        ),
        scratch_types=[
            pltpu.VMEM((NSLOT,), jnp.int32),
            pltpu.VMEM((NSLOT,), jnp.float32),
            pltpu.VMEM((2 * S,), jnp.int32),
            pltpu.VMEM((2 * S,), jnp.float32),
        ],
        compiler_params=pltpu.CompilerParams(needs_layout_passes=False),
    )
    def k(slots_hbm, wts_hbm, ss_hbm, ws_hbm, sv, wv, slv, wtv):
        wid = lax.axis_index("s") * 2 + lax.axis_index("c")

        @pl.when(wid == 0)
        def _():
            pltpu.sync_copy(slots_hbm, slv)

            def init(i, carry):
                sv[pl.ds(i * 16, 16)] = jnp.full((16,), ZROW, jnp.int32)
                return carry

            lax.fori_loop(0, NSLOT // 16, init, 0)

            def scat(i, carry):
                idx = slv[pl.ds(i * 16, 16)]
                tok = (lax.iota(jnp.int32, 16) + i * 16) & (S - 1)
                plsc.store_scatter(sv, [idx], tok)
                return carry

            lax.fori_loop(0, 2 * S // 16, scat, 0)
            pltpu.sync_copy(sv, ss_hbm)

        @pl.when(wid == 1)
        def _():
            pltpu.sync_copy(slots_hbm, slv)
            pltpu.sync_copy(wts_hbm, wtv)

            def initw(i, carry):
                wv[pl.ds(i * 16, 16)] = jnp.zeros((16,), jnp.float32)
                return carry

            lax.fori_loop(0, NSLOT // 16, initw, 0)

            def scatw(i, carry):
                idx = slv[pl.ds(i * 16, 16)]
                w = wtv[pl.ds(i * 16, 16)]
                plsc.store_scatter(wv, [idx], w)
                return carry

            lax.fori_loop(0, 2 * S // 16, scatw, 0)
            pltpu.sync_copy(wv, ws_hbm)

    return k(slots, wts)


def _gather_call(src, idx, chunk=32):
    n = idx.shape[0]
    per_w = n // NW
    nch = per_w // chunk
    mesh = plsc.VectorSubcoreMesh(core_axis_name="c", subcore_axis_name="s")

    @functools.partial(
        pl.kernel,
        mesh=mesh,
        out_type=jax.ShapeDtypeStruct((n, M), jnp.float32),
        scratch_types=[
            pltpu.VMEM((per_w,), jnp.int32),
            pltpu.VMEM((chunk, M), jnp.float32),
            pltpu.VMEM((chunk, M), jnp.float32),
            pltpu.VMEM((chunk, M), jnp.float32),
            pltpu.SemaphoreType.DMA,
            pltpu.SemaphoreType.DMA,
            pltpu.SemaphoreType.DMA,
            pltpu.SemaphoreType.DMA,
            pltpu.SemaphoreType.DMA,
            pltpu.SemaphoreType.DMA,
        ],
    )
    def k(src_hbm, idx_hbm, out_hbm, idxv, b0, b1, b2, g0, g1, g2, o0, o1, o2):
        wid = lax.axis_index("s") * 2 + lax.axis_index("c")
        base = wid * per_w
        pltpu.sync_copy(idx_hbm.at[pl.ds(base, per_w)], idxv)
        bufs = [(b0, g0, o0), (b1, g1, o1), (b2, g2, o2)]
        nbuf = 3
        gathers = [None] * nch
        writes = [None] * nch
        for g in range(nch):
            b, gsem, _ = bufs[g % nbuf]
            if g >= nbuf:
                writes[g - nbuf].wait()
            gathers[g] = pltpu.async_copy(
                src_hbm.at[idxv.at[pl.ds(g * chunk, chunk)]], b, gsem)
            if g >= 1:
                pb, _, posem = bufs[(g - 1) % nbuf]
                gathers[g - 1].wait()
                writes[g - 1] = pltpu.async_copy(
                    pb, out_hbm.at[pl.ds(base + (g - 1) * chunk, chunk)], posem)
        gathers[nch - 1].wait()
        lb, _, losem = bufs[(nch - 1) % nbuf]
        writes[nch - 1] = pltpu.async_copy(
            lb, out_hbm.at[pl.ds(base + (nch - 1) * chunk, chunk)], losem)
        for t in range(max(0, nch - nbuf), nch):
            writes[t].wait()

    return k(src, idx)


def _ffn_body(d_ref, w1_ref, w2_ref, ws_ref, o_ref):
    f = pl.program_id(1)

    @pl.when(f == 0)
    def _():
        o_ref[...] = jnp.zeros_like(o_ref)

    db = d_ref[...].astype(jnp.bfloat16)
    w1b = w1_ref[0].astype(jnp.bfloat16)
    h = jnp.dot(db, w1b, preferred_element_type=jnp.float32)
    hb = jnp.maximum(h, 0.0).astype(jnp.bfloat16)
    w2b = w2_ref[0].astype(jnp.bfloat16)
    o_ref[...] += jnp.dot(hb, w2b, preferred_element_type=jnp.float32)

    @pl.when(f == FSTEPS - 1)
    def _():
        o_ref[...] *= ws_ref[...]


_ffn = pl.pallas_call(
    _ffn_body,
    grid=(E, FSTEPS),
    in_specs=[
        pl.BlockSpec((C, M), lambda e, f: (e, 0)),
        pl.BlockSpec((1, M, FBLK), lambda e, f: (e, 0, f)),
        pl.BlockSpec((1, FBLK, M), lambda e, f: (e, f, 0)),
        pl.BlockSpec((C, 1), lambda e, f: (e, 0)),
    ],
    out_specs=pl.BlockSpec((C, M), lambda e, f: (e, 0)),
    out_shape=jax.ShapeDtypeStruct((E * C, M), jnp.float32),
    compiler_params=pltpu.CompilerParams(
        dimension_semantics=("parallel", "arbitrary")),
)


def _add_body(a_ref, b_ref, va_ref, vb_ref, o_ref):
    o_ref[...] = a_ref[...] * va_ref[...] + b_ref[...] * vb_ref[...]


_combine_add = pl.pallas_call(
    _add_body,
    grid=(S // C,),
    in_specs=[
        pl.BlockSpec((C, M), lambda i: (i, 0)),
        pl.BlockSpec((C, M), lambda i: (i + S // C, 0)),
        pl.BlockSpec((C, 1), lambda i: (i, 0)),
        pl.BlockSpec((C, 1), lambda i: (i, 0)),
    ],
    out_specs=pl.BlockSpec((C, M), lambda i: (i, 0)),
    out_shape=jax.ShapeDtypeStruct((S, M), jnp.float32),
)


def kernel(input, wg, w1, w2):
    x = input
    s0, s1, c0, c1, w0g, w1g, va, vb, ll = _router(x, wg)
    slots = jnp.concatenate([s0.reshape(S), s1.reshape(S)])
    cslots = jnp.concatenate([c0.reshape(S), c1.reshape(S)])
    wts = jnp.concatenate([w0g.reshape(S), w1g.reshape(S)])
    slot_src, wslot = _scatter_call(slots, wts)
    disp = _gather_call(x, slot_src[:E * C])
    eo = _ffn(disp, w1, w2, wslot[:E * C].reshape(E * C, 1))
    ab = _gather_call(eo, cslots)
    y = _combine_add(ab, ab, va, vb)
    return y, ll.reshape(())
